# baseline (device time: 181401 ns/iter reference)
import jax
import jax.numpy as jnp
from jax import lax
from jax.experimental import pallas as pl
from jax.experimental.pallas import tpu as pltpu

BLK = 1024
CHUNK = 128
NCH = BLK // CHUNK
NH = NCH // 2
NST = 4


def kernel(partial, resid, gamma):
    _, M, D = partial.shape
    partial2 = partial.reshape(M, D)
    gamma2 = gamma.reshape(1, D)

    def body(pa_ref, re_ref, ga_ref, out_ref,
             lv, send_x, recv_x, g_y, g_z, g_diag,
             ldma_sems, st_sems,
             sx_s, sx_r, sy1_s, sy1_r, sz1_s, sz1_r,
             g2y_s, g2y_r, g2z_s, g2z_r,
             fxy_s, fxy_r, fxz_s, fxz_r):
        my_x = lax.axis_index("x")
        my_y = lax.axis_index("y")
        my_z = lax.axis_index("z")
        nbr_x = (1 - my_x, my_y, my_z)
        nbr_y = (my_x, 1 - my_y, my_z)
        nbr_z = (my_x, my_y, 1 - my_z)

        q = 2 * my_y + my_z
        q_y = 2 * (1 - my_y) + my_z
        q_z = 2 * my_y + (1 - my_z)
        q_d = 2 * (1 - my_y) + (1 - my_z)
        row0 = q * BLK

        g_own = send_x

        bsem = pltpu.get_barrier_semaphore()
        for nbr in (nbr_x, nbr_y, nbr_z):
            pl.semaphore_signal(bsem, inc=1, device_id=nbr,
                                device_id_type=pl.DeviceIdType.MESH)
        pl.semaphore_wait(bsem, 3)

        def chunk(c):
            return pl.ds(c * CHUNK, CHUNK)

        def load(ref, src_row, slot):
            cp = pltpu.make_async_copy(
                ref.at[pl.ds(src_row, CHUNK)], lv.at[slot],
                ldma_sems.at[slot])
            cp.start()
            return cp

        pending = [None] * NST

        def emit_store(src, dst_row):
            slot = emit_store.n % NST
            emit_store.n += 1
            if pending[slot] is not None:
                pending[slot].wait()
            cp = pltpu.make_async_copy(
                src, out_ref.at[pl.ds(dst_row, CHUNK)], st_sems.at[slot])
            cp.start()
            pending[slot] = cp
        emit_store.n = 0

        rx = []
        ld = load(pa_ref, row0, 0)
        for c in range(NCH):
            nxt = load(pa_ref, row0 + (c + 1) * CHUNK, (c + 1) % 2) \
                if c + 1 < NCH else None
            ld.wait()
            send_x[chunk(c)] = lv[c % 2].astype(jnp.bfloat16)
            r = pltpu.make_async_remote_copy(
                src_ref=send_x.at[chunk(c)], dst_ref=recv_x.at[chunk(c)],
                send_sem=sx_s.at[c], recv_sem=sx_r.at[c],
                device_id=nbr_x, device_id_type=pl.DeviceIdType.MESH)
            r.start()
            rx.append(r)
            ld = nxt

        is0 = my_x == 0
        is1 = my_x == 1
        g1y, g1z = [], []
        ld = load(re_ref, row0, 0)
        for c in range(NCH):
            nxt = load(re_ref, row0 + (c + 1) * CHUNK, (c + 1) % 2) \
                if c + 1 < NCH else None
            rx[c].wait_recv()
            ld.wait()
            y = (send_x[chunk(c)].astype(jnp.float32)
                 + recv_x[chunk(c)].astype(jnp.float32)
                 + lv[c % 2])
            ms = jnp.mean(y * y, axis=-1, keepdims=True) + 1e-6
            o = (y * lax.rsqrt(ms) * ga_ref[...]).astype(jnp.bfloat16)
            rx[c].wait_send()
            g_own[chunk(c)] = o
            emit_store(g_own.at[chunk(c)], row0 + c * CHUNK)
            ry = pltpu.make_async_remote_copy(
                src_ref=g_own.at[chunk(c)], dst_ref=g_y.at[chunk(c)],
                send_sem=sy1_s.at[c], recv_sem=sy1_r.at[c],
                device_id=nbr_y, device_id_type=pl.DeviceIdType.MESH)
            rz = pltpu.make_async_remote_copy(
                src_ref=g_own.at[chunk(c)], dst_ref=g_z.at[chunk(c)],
                send_sem=sz1_s.at[c], recv_sem=sz1_r.at[c],
                device_id=nbr_z, device_id_type=pl.DeviceIdType.MESH)
            if c in (0, 1):
                pl.when(my_x != c)(ry.start)
            else:
                ry.start()
            if c in (4, 5):
                pl.when(4 + my_x != c)(rz.start)
            else:
                rz.start()
            g1y.append(ry)
            g1z.append(rz)
            ld = nxt

        fy = [pltpu.make_async_remote_copy(
            src_ref=g_y.at[chunk(c)], dst_ref=g_y.at[chunk(c)],
            send_sem=fxy_s, recv_sem=fxy_r,
            device_id=nbr_x, device_id_type=pl.DeviceIdType.MESH)
            for c in (0, 1)]
        fz = [pltpu.make_async_remote_copy(
            src_ref=g_z.at[chunk(c)], dst_ref=g_z.at[chunk(c)],
            send_sem=fxz_s, recv_sem=fxz_r,
            device_id=nbr_x, device_id_type=pl.DeviceIdType.MESH)
            for c in (4, 5)]

        @pl.when(is1)
        def _():
            g1y[0].wait_recv()
            fy[0].start()
            g1z[4].wait_recv()
            fz[0].start()

        @pl.when(is0)
        def _():
            g1y[1].wait_recv()
            fy[1].start()
            g1z[5].wait_recv()
            fz[1].start()

        g2y = []
        for k in range(NH):
            g1z[k].wait_recv()
            r = pltpu.make_async_remote_copy(
                src_ref=g_z.at[chunk(k)], dst_ref=g_diag.at[chunk(k)],
                send_sem=g2y_s.at[k], recv_sem=g2y_r.at[k],
                device_id=nbr_y, device_id_type=pl.DeviceIdType.MESH)
            r.start()
            g2y.append(r)
            emit_store(g_z.at[chunk(k)], q_z * BLK + k * CHUNK)
        g2z = []
        for k in range(NH, NCH):
            g1y[k].wait_recv()
            r = pltpu.make_async_remote_copy(
                src_ref=g_y.at[chunk(k)], dst_ref=g_diag.at[chunk(k)],
                send_sem=g2z_s.at[k - NH], recv_sem=g2z_r.at[k - NH],
                device_id=nbr_z, device_id_type=pl.DeviceIdType.MESH)
            r.start()
            g2z.append(r)
            emit_store(g_y.at[chunk(k)], q_y * BLK + k * CHUNK)

        for k in (6, 7):
            g1z[k].wait_recv()
            emit_store(g_z.at[chunk(k)], q_z * BLK + k * CHUNK)
        for k in (2, 3):
            g1y[k].wait_recv()
            emit_store(g_y.at[chunk(k)], q_y * BLK + k * CHUNK)
        pl.when(is0)(fz[0].wait_recv)
        pl.when(is1)(fz[1].wait_recv)
        emit_store(g_z.at[chunk(4)], q_z * BLK + 4 * CHUNK)
        emit_store(g_z.at[chunk(5)], q_z * BLK + 5 * CHUNK)
        pl.when(is0)(fy[0].wait_recv)
        pl.when(is1)(fy[1].wait_recv)
        emit_store(g_y.at[chunk(0)], q_y * BLK + 0 * CHUNK)
        emit_store(g_y.at[chunk(1)], q_y * BLK + 1 * CHUNK)
        for k in range(NH):
            g2y[k].wait_recv()
            emit_store(g_diag.at[chunk(k)], q_d * BLK + k * CHUNK)
        for k in range(NH, NCH):
            g2z[k - NH].wait_recv()
            emit_store(g_diag.at[chunk(k)], q_d * BLK + k * CHUNK)

        for c in (0, 1):
            pl.when(my_x != c)(g1y[c].wait_send)
        for c in (4, 5):
            pl.when(4 + my_x != c)(g1z[c].wait_send)
        for r in g1y[2:] + g1z[:4] + g1z[6:] + g2y + g2z:
            r.wait_send()
        pl.when(is1)(fy[0].wait_send)
        pl.when(is1)(fz[0].wait_send)
        pl.when(is0)(fy[1].wait_send)
        pl.when(is0)(fz[1].wait_send)
        for slot in range(NST):
            if pending[slot] is not None:
                pending[slot].wait()

    return pl.pallas_call(
        body,
        out_shape=jax.ShapeDtypeStruct((M, D), jnp.bfloat16),
        in_specs=[
            pl.BlockSpec(memory_space=pl.ANY),
            pl.BlockSpec(memory_space=pl.ANY),
            pl.BlockSpec(memory_space=pltpu.VMEM),
        ],
        out_specs=pl.BlockSpec(memory_space=pl.ANY),
        scratch_shapes=[
            pltpu.VMEM((2, CHUNK, D), jnp.float32),
            pltpu.VMEM((BLK, D), jnp.bfloat16),
            pltpu.VMEM((BLK, D), jnp.bfloat16),
            pltpu.VMEM((BLK, D), jnp.bfloat16),
            pltpu.VMEM((BLK, D), jnp.bfloat16),
            pltpu.VMEM((BLK, D), jnp.bfloat16),
            pltpu.SemaphoreType.DMA((2,)),
            pltpu.SemaphoreType.DMA((NST,)),
            pltpu.SemaphoreType.DMA((NCH,)),
            pltpu.SemaphoreType.DMA((NCH,)),
            pltpu.SemaphoreType.DMA((NCH,)),
            pltpu.SemaphoreType.DMA((NCH,)),
            pltpu.SemaphoreType.DMA((NCH,)),
            pltpu.SemaphoreType.DMA((NCH,)),
            pltpu.SemaphoreType.DMA((NH,)),
            pltpu.SemaphoreType.DMA((NH,)),
            pltpu.SemaphoreType.DMA((NH,)),
            pltpu.SemaphoreType.DMA((NH,)),
            pltpu.SemaphoreType.DMA,
            pltpu.SemaphoreType.DMA,
            pltpu.SemaphoreType.DMA,
            pltpu.SemaphoreType.DMA,
        ],
        compiler_params=pltpu.CompilerParams(
            collective_id=0,
            vmem_limit_bytes=56 * 1024 * 1024,
        ),
    )(partial2, resid, gamma2)


# device time: 180180 ns/iter; 1.0068x vs baseline; 1.0068x over previous
import jax
import jax.numpy as jnp
from jax import lax
from jax.experimental import pallas as pl
from jax.experimental.pallas import tpu as pltpu

BLK = 1024
CHUNK = 128
NCH = BLK // CHUNK
NH = NCH // 2
NST = 4


def kernel(partial, resid, gamma):
    _, M, D = partial.shape
    partial2 = partial.reshape(M, D)
    gamma2 = gamma.reshape(1, D)

    def body(pa_ref, re_ref, ga_ref, out_ref,
             lv, send_x, recv_x, g_y, g_z, g_diag,
             ldma_sems, st_sems,
             sx_s, sx_r, sy1_s, sy1_r, sz1_s, sz1_r,
             g2y_s, g2y_r, g2z_s, g2z_r):
        my_x = lax.axis_index("x")
        my_y = lax.axis_index("y")
        my_z = lax.axis_index("z")
        nbr_x = (1 - my_x, my_y, my_z)
        nbr_y = (my_x, 1 - my_y, my_z)
        nbr_z = (my_x, my_y, 1 - my_z)

        q = 2 * my_y + my_z
        q_y = 2 * (1 - my_y) + my_z
        q_z = 2 * my_y + (1 - my_z)
        q_d = 2 * (1 - my_y) + (1 - my_z)
        row0 = q * BLK

        g_own = send_x

        bsem = pltpu.get_barrier_semaphore()
        for nbr in (nbr_x, nbr_y, nbr_z):
            pl.semaphore_signal(bsem, inc=1, device_id=nbr,
                                device_id_type=pl.DeviceIdType.MESH)
        pl.semaphore_wait(bsem, 3)

        def chunk(c):
            return pl.ds(c * CHUNK, CHUNK)

        def load(ref, src_row, slot):
            cp = pltpu.make_async_copy(
                ref.at[pl.ds(src_row, CHUNK)], lv.at[slot],
                ldma_sems.at[slot])
            cp.start()
            return cp

        pending = [None] * NST

        def emit_store(src, dst_row):
            slot = emit_store.n % NST
            emit_store.n += 1
            if pending[slot] is not None:
                pending[slot].wait()
            cp = pltpu.make_async_copy(
                src, out_ref.at[pl.ds(dst_row, CHUNK)], st_sems.at[slot])
            cp.start()
            pending[slot] = cp
        emit_store.n = 0

        rx = []
        ld = load(pa_ref, row0, 0)
        for c in range(NCH):
            nxt = load(pa_ref, row0 + (c + 1) * CHUNK, (c + 1) % 2) \
                if c + 1 < NCH else None
            ld.wait()
            send_x[chunk(c)] = lv[c % 2].astype(jnp.bfloat16)
            r = pltpu.make_async_remote_copy(
                src_ref=send_x.at[chunk(c)], dst_ref=recv_x.at[chunk(c)],
                send_sem=sx_s.at[c], recv_sem=sx_r.at[c],
                device_id=nbr_x, device_id_type=pl.DeviceIdType.MESH)
            r.start()
            rx.append(r)
            ld = nxt

        g1y, g1z = [], []
        ld = load(re_ref, row0, 0)
        for c in range(NCH):
            nxt = load(re_ref, row0 + (c + 1) * CHUNK, (c + 1) % 2) \
                if c + 1 < NCH else None
            rx[c].wait_recv()
            ld.wait()
            y = (send_x[chunk(c)].astype(jnp.float32)
                 + recv_x[chunk(c)].astype(jnp.float32)
                 + lv[c % 2])
            ms = jnp.mean(y * y, axis=-1, keepdims=True) + 1e-6
            o = (y * lax.rsqrt(ms) * ga_ref[...]).astype(jnp.bfloat16)
            rx[c].wait_send()
            g_own[chunk(c)] = o
            emit_store(g_own.at[chunk(c)], row0 + c * CHUNK)
            ry = pltpu.make_async_remote_copy(
                src_ref=g_own.at[chunk(c)], dst_ref=g_y.at[chunk(c)],
                send_sem=sy1_s.at[c], recv_sem=sy1_r.at[c],
                device_id=nbr_y, device_id_type=pl.DeviceIdType.MESH)
            rz = pltpu.make_async_remote_copy(
                src_ref=g_own.at[chunk(c)], dst_ref=g_z.at[chunk(c)],
                send_sem=sz1_s.at[c], recv_sem=sz1_r.at[c],
                device_id=nbr_z, device_id_type=pl.DeviceIdType.MESH)
            ry.start()
            rz.start()
            g1y.append(ry)
            g1z.append(rz)
            ld = nxt

        g2y = []
        for k in range(NH):
            g1z[k].wait_recv()
            r = pltpu.make_async_remote_copy(
                src_ref=g_z.at[chunk(k)], dst_ref=g_diag.at[chunk(k)],
                send_sem=g2y_s.at[k], recv_sem=g2y_r.at[k],
                device_id=nbr_y, device_id_type=pl.DeviceIdType.MESH)
            r.start()
            g2y.append(r)
            emit_store(g_z.at[chunk(k)], q_z * BLK + k * CHUNK)
        g2z = []
        for k in range(NH, NCH):
            g1y[k].wait_recv()
            r = pltpu.make_async_remote_copy(
                src_ref=g_y.at[chunk(k)], dst_ref=g_diag.at[chunk(k)],
                send_sem=g2z_s.at[k - NH], recv_sem=g2z_r.at[k - NH],
                device_id=nbr_z, device_id_type=pl.DeviceIdType.MESH)
            r.start()
            g2z.append(r)
            emit_store(g_y.at[chunk(k)], q_y * BLK + k * CHUNK)

        for k in range(NH, NCH):
            g1z[k].wait_recv()
            emit_store(g_z.at[chunk(k)], q_z * BLK + k * CHUNK)
        for k in range(NH):
            g1y[k].wait_recv()
            emit_store(g_y.at[chunk(k)], q_y * BLK + k * CHUNK)
        for k in range(NH):
            g2y[k].wait_recv()
            emit_store(g_diag.at[chunk(k)], q_d * BLK + k * CHUNK)
        for k in range(NH, NCH):
            g2z[k - NH].wait_recv()
            emit_store(g_diag.at[chunk(k)], q_d * BLK + k * CHUNK)

        for r in g1y + g1z + g2y + g2z:
            r.wait_send()
        for slot in range(NST):
            if pending[slot] is not None:
                pending[slot].wait()

    return pl.pallas_call(
        body,
        out_shape=jax.ShapeDtypeStruct((M, D), jnp.bfloat16),
        in_specs=[
            pl.BlockSpec(memory_space=pl.ANY),
            pl.BlockSpec(memory_space=pl.ANY),
            pl.BlockSpec(memory_space=pltpu.VMEM),
        ],
        out_specs=pl.BlockSpec(memory_space=pl.ANY),
        scratch_shapes=[
            pltpu.VMEM((2, CHUNK, D), jnp.float32),
            pltpu.VMEM((BLK, D), jnp.bfloat16),
            pltpu.VMEM((BLK, D), jnp.bfloat16),
            pltpu.VMEM((BLK, D), jnp.bfloat16),
            pltpu.VMEM((BLK, D), jnp.bfloat16),
            pltpu.VMEM((BLK, D), jnp.bfloat16),
            pltpu.SemaphoreType.DMA((2,)),
            pltpu.SemaphoreType.DMA((NST,)),
            pltpu.SemaphoreType.DMA((NCH,)),
            pltpu.SemaphoreType.DMA((NCH,)),
            pltpu.SemaphoreType.DMA((NCH,)),
            pltpu.SemaphoreType.DMA((NCH,)),
            pltpu.SemaphoreType.DMA((NCH,)),
            pltpu.SemaphoreType.DMA((NCH,)),
            pltpu.SemaphoreType.DMA((NH,)),
            pltpu.SemaphoreType.DMA((NH,)),
            pltpu.SemaphoreType.DMA((NH,)),
            pltpu.SemaphoreType.DMA((NH,)),
        ],
        compiler_params=pltpu.CompilerParams(
            collective_id=0,
            vmem_limit_bytes=56 * 1024 * 1024,
        ),
    )(partial2, resid, gamma2)
